# baseline (speedup 1.0000x reference)
import functools

import jax
import jax.numpy as jnp
from jax.experimental import pallas as pl
from jax.experimental.pallas import tpu as pltpu


def _fused_kernel(xp_ref, wsr_ref, bsr_ref, wq_ref, bq_ref,
                  wkv_ref, bkv_ref, wp_ref, bp_ref, o_ref, x_sc,
                  *, Hs, Ws, sr, C, N, eps, bb):
    f32 = jnp.float32

    for i in range(bb):
        cols = []
        for dh in range(sr):
            rows = [xp_ref[i, hs, dh] for hs in range(Hs)]
            cols.append(jnp.concatenate(rows, axis=0))
        pmat = jnp.concatenate(cols, axis=1)

        for hs in range(Hs):
            for dh in range(sr):
                slab = xp_ref[i, hs, dh]
                pieces = [jax.lax.slice(slab, (0, dw * C), (Ws, (dw + 1) * C))
                          for dw in range(sr)]
                st = jnp.stack(pieces, axis=1)
                x_sc[i, pl.ds((hs * sr + dh) * Ws, Ws)] = st

        y = jnp.dot(pmat, wsr_ref[...], preferred_element_type=f32) + bsr_ref[...]
        mu = jnp.mean(y, axis=-1, keepdims=True)
        yc = y - mu
        var = jnp.mean(yc * yc, axis=-1, keepdims=True)
        xs = yc * jax.lax.rsqrt(var + eps)

        kv = jnp.dot(xs, wkv_ref[...], preferred_element_type=f32) + bkv_ref[...]
        k = kv[:, :C]
        v = kv[:, C:]
        kbq = jnp.dot(k, bq_ref[...], preferred_element_type=f32)

        xt = x_sc[i].reshape(N, C)
        qT = jax.lax.dot_general(wq_ref[...], xt, (((0,), (1,)), ((), ())),
                                 preferred_element_type=f32)
        s = jax.lax.dot_general(k, qT, (((1,), (0,)), ((), ())),
                                preferred_element_type=f32) + kbq
        m = jnp.max(s, axis=0, keepdims=True)
        p = jnp.exp(s - m)
        l = jnp.sum(p, axis=0, keepdims=True)
        oT = jax.lax.dot_general(v, p, (((0,), (0,)), ((), ())),
                                 preferred_element_type=f32)
        oT = oT * pl.reciprocal(l, approx=True)
        res = jax.lax.dot_general(oT, wp_ref[...], (((0,), (0,)), ((), ())),
                                  preferred_element_type=f32) + bp_ref[...]
        o_ref[i] = res.astype(o_ref.dtype)


def kernel(x, wq_t, bq, wk_t, bk, wv_t, bv, wp_t, bp, wsr_t, bsr, ln_g, ln_b):
    B, N, C = x.shape
    H = W = 56
    sr = 8
    Hs, Ws = H // sr, W // sr
    scale = float(C) ** -0.5

    xp = x.reshape(B, Hs, sr, Ws, sr * C)
    wsr_r = wsr_t.reshape(C, sr, sr, C).transpose(1, 2, 0, 3).reshape(sr * sr * C, C)
    wq_s = wq_t * scale
    bq_s = (bq * scale).reshape(C, 1)
    wkv = jnp.concatenate([wk_t, wv_t], axis=1) * ln_g.reshape(C, 1)
    bkv = (jnp.concatenate([bk, bv])
           + jnp.dot(ln_b, jnp.concatenate([wk_t, wv_t], axis=1),
                     precision=jax.lax.Precision.HIGHEST)).reshape(1, 2 * C)

    bb = 4 if B % 4 == 0 else 1
    body = functools.partial(_fused_kernel, Hs=Hs, Ws=Ws, sr=sr, C=C, N=N,
                             eps=1e-5, bb=bb)

    return pl.pallas_call(
        body,
        out_shape=jax.ShapeDtypeStruct((B, N, C), x.dtype),
        grid=(B // bb,),
        in_specs=[
            pl.BlockSpec((bb, Hs, sr, Ws, sr * C), lambda b: (b, 0, 0, 0, 0)),
            pl.BlockSpec((sr * sr * C, C), lambda b: (0, 0)),
            pl.BlockSpec((1, C), lambda b: (0, 0)),
            pl.BlockSpec((C, C), lambda b: (0, 0)),
            pl.BlockSpec((C, 1), lambda b: (0, 0)),
            pl.BlockSpec((C, 2 * C), lambda b: (0, 0)),
            pl.BlockSpec((1, 2 * C), lambda b: (0, 0)),
            pl.BlockSpec((C, C), lambda b: (0, 0)),
            pl.BlockSpec((1, C), lambda b: (0, 0)),
        ],
        out_specs=pl.BlockSpec((bb, N, C), lambda b: (b, 0, 0)),
        scratch_shapes=[
            pltpu.VMEM((bb, N // sr, sr, C), jnp.float32),
        ],
        compiler_params=pltpu.CompilerParams(
            dimension_semantics=("parallel",),
            vmem_limit_bytes=64 * 1024 * 1024,
        ),
    )(xp, wsr_r, bsr.reshape(1, C), wq_s, bq_s, wkv, bkv, wp_t,
      bp.reshape(1, C))


# speedup vs baseline: 1.0193x; 1.0193x over previous
import functools

import jax
import jax.numpy as jnp
from jax.experimental import pallas as pl
from jax.experimental.pallas import tpu as pltpu


def _fused_kernel(xp_ref, wsr_ref, bsr_ref, wqbd_ref, bqbd_ref,
                  wkg_ref, bkt_ref, wvg_ref, bv_ref, sb2_ref,
                  wpbd_ref, bpbd_ref, o_ref, kbd_sc, vbd_sc,
                  *, Hs, Ws, sr, C, eps, bb):
    f32 = jnp.float32
    Nk = Hs * Ws
    NW = Hs * sr * Ws

    for i in range(bb):
        cols = []
        for dh in range(sr):
            rows = [xp_ref[i, hs, dh] for hs in range(Hs)]
            cols.append(jnp.concatenate(rows, axis=0))
        pmat = jnp.concatenate(cols, axis=1)

        xw = jnp.concatenate(
            [xp_ref[i, hs, dh] for hs in range(Hs) for dh in range(sr)],
            axis=0)

        y = jnp.dot(pmat, wsr_ref[...], preferred_element_type=f32) + bsr_ref[...]
        mu = jnp.mean(y, axis=-1, keepdims=True)
        yc = y - mu
        var = jnp.mean(yc * yc, axis=-1, keepdims=True)
        xs = yc * jax.lax.rsqrt(var + eps)

        kT = jax.lax.dot_general(wkg_ref[...], xs, (((0,), (1,)), ((), ())),
                                 preferred_element_type=f32) + bkt_ref[...]
        v = jnp.dot(xs, wvg_ref[...], preferred_element_type=f32) + bv_ref[...]

        kbd_sc[...] = jnp.zeros((sr * C, NW), f32)
        vbd_sc[...] = jnp.zeros((NW, sr * C), f32)
        for dw in range(sr):
            kbd_sc[pl.ds(dw * C, C), pl.ds(dw * Nk, Nk)] = kT
            vbd_sc[pl.ds(dw * Nk, Nk), pl.ds(dw * C, C)] = v

        qw = jnp.dot(xw, wqbd_ref[...], preferred_element_type=f32) + bqbd_ref[...]
        s = jnp.dot(qw, kbd_sc[...], preferred_element_type=f32)
        p = jnp.exp(s - jnp.max(s))
        lsp = jnp.dot(p, sb2_ref[...], preferred_element_type=f32)
        ow = jnp.dot(p, vbd_sc[...], preferred_element_type=f32)
        oN = ow * pl.reciprocal(lsp, approx=True)
        res = jnp.dot(oN, wpbd_ref[...], preferred_element_type=f32) + bpbd_ref[...]
        o_ref[i] = res.astype(o_ref.dtype)


def kernel(x, wq_t, bq, wk_t, bk, wv_t, bv, wp_t, bp, wsr_t, bsr, ln_g, ln_b):
    B, N, C = x.shape
    H = W = 56
    sr = 8
    Hs, Ws = H // sr, W // sr
    Nk = Hs * Ws
    NW = N // sr
    scale = float(C) ** -0.5
    f32 = jnp.float32

    xp = x.reshape(B, Hs, sr, Ws, sr * C)
    wsr_r = wsr_t.reshape(C, sr, sr, C).transpose(1, 2, 0, 3).reshape(sr * sr * C, C)

    hi = jax.lax.Precision.HIGHEST
    eye = jnp.eye(sr, dtype=f32)
    wqbd = jnp.kron(eye, wq_t * scale)
    bqbd = jnp.tile((bq * scale).reshape(1, C), (1, sr))
    wkg = wk_t * ln_g.reshape(C, 1)
    bkt = (bk + jnp.dot(ln_b, wk_t, precision=hi)).reshape(C, 1)
    wvg = wv_t * ln_g.reshape(C, 1)
    bv_r = (bv + jnp.dot(ln_b, wv_t, precision=hi)).reshape(1, C)
    sb2 = jnp.kron(eye, jnp.ones((Nk, C), f32))
    wpbd = jnp.kron(eye, wp_t)
    bpbd = jnp.tile(bp.reshape(1, C), (1, sr))

    bb = 4 if B % 4 == 0 else 1
    body = functools.partial(_fused_kernel, Hs=Hs, Ws=Ws, sr=sr, C=C,
                             eps=1e-5, bb=bb)

    out = pl.pallas_call(
        body,
        out_shape=jax.ShapeDtypeStruct((B, NW, sr * C), x.dtype),
        grid=(B // bb,),
        in_specs=[
            pl.BlockSpec((bb, Hs, sr, Ws, sr * C), lambda b: (b, 0, 0, 0, 0)),
            pl.BlockSpec((sr * sr * C, C), lambda b: (0, 0)),
            pl.BlockSpec((1, C), lambda b: (0, 0)),
            pl.BlockSpec((sr * C, sr * C), lambda b: (0, 0)),
            pl.BlockSpec((1, sr * C), lambda b: (0, 0)),
            pl.BlockSpec((C, C), lambda b: (0, 0)),
            pl.BlockSpec((C, 1), lambda b: (0, 0)),
            pl.BlockSpec((C, C), lambda b: (0, 0)),
            pl.BlockSpec((1, C), lambda b: (0, 0)),
            pl.BlockSpec((NW, sr * C), lambda b: (0, 0)),
            pl.BlockSpec((sr * C, sr * C), lambda b: (0, 0)),
            pl.BlockSpec((1, sr * C), lambda b: (0, 0)),
        ],
        out_specs=pl.BlockSpec((bb, NW, sr * C), lambda b: (b, 0, 0)),
        scratch_shapes=[
            pltpu.VMEM((sr * C, NW), jnp.float32),
            pltpu.VMEM((NW, sr * C), jnp.float32),
        ],
        compiler_params=pltpu.CompilerParams(
            dimension_semantics=("parallel",),
            vmem_limit_bytes=64 * 1024 * 1024,
        ),
    )(xp, wsr_r, bsr.reshape(1, C), wqbd, bqbd, wkg, bkt, wvg, bv_r,
      sb2, wpbd, bpbd)
    return out.reshape(B, N, C)


# speedup vs baseline: 1.2087x; 1.1858x over previous
import functools

import jax
import jax.numpy as jnp
from jax.experimental import pallas as pl
from jax.experimental.pallas import tpu as pltpu


def _fused_kernel(xn_ref, xp_ref, wsr_ref, bsr_ref, wq_ref, bq_ref,
                  wkv_ref, bkv_ref, wp_ref, bp_ref, o_ref,
                  *, Hs, sr, C, N, tq, eps, bb):
    f32 = jnp.float32

    for i in range(bb):
        cols = []
        for dh in range(sr):
            rows = [xp_ref[i, hs, dh] for hs in range(Hs)]
            cols.append(jnp.concatenate(rows, axis=0))
        pmat = jnp.concatenate(cols, axis=1)

        y = jnp.dot(pmat, wsr_ref[...], preferred_element_type=f32) + bsr_ref[...]
        mu = jnp.mean(y, axis=-1, keepdims=True)
        yc = y - mu
        var = jnp.mean(yc * yc, axis=-1, keepdims=True)
        xs = yc * jax.lax.rsqrt(var + eps)

        kv = jnp.dot(xs, wkv_ref[...], preferred_element_type=f32) + bkv_ref[...]
        k = kv[:, :C]
        v = kv[:, C:]
        kbq = jnp.dot(k, bq_ref[...], preferred_element_type=f32)

        n_qt = N // tq
        for qt in range(n_qt):
            xt = xn_ref[i, pl.ds(qt * tq, tq), :]
            qT = jax.lax.dot_general(wq_ref[...], xt, (((0,), (1,)), ((), ())),
                                     preferred_element_type=f32)
            s = jax.lax.dot_general(k, qT, (((1,), (0,)), ((), ())),
                                    preferred_element_type=f32) + kbq
            m = jnp.max(s, axis=0, keepdims=True)
            p = jnp.exp(s - m)
            l = jnp.sum(p, axis=0, keepdims=True)
            oT = jax.lax.dot_general(v, p, (((0,), (0,)), ((), ())),
                                     preferred_element_type=f32)
            oT = oT * pl.reciprocal(l, approx=True)
            res = jax.lax.dot_general(oT, wp_ref[...], (((0,), (0,)), ((), ())),
                                      preferred_element_type=f32) + bp_ref[...]
            o_ref[i, pl.ds(qt * tq, tq), :] = res.astype(o_ref.dtype)


def kernel(x, wq_t, bq, wk_t, bk, wv_t, bv, wp_t, bp, wsr_t, bsr, ln_g, ln_b):
    B, N, C = x.shape
    H = W = 56
    sr = 8
    Hs, Ws = H // sr, W // sr
    scale = float(C) ** -0.5
    tq = 3136

    xp = x.reshape(B, Hs, sr, Ws, sr * C)
    wsr_r = wsr_t.reshape(C, sr, sr, C).transpose(1, 2, 0, 3).reshape(sr * sr * C, C)
    wq_s = wq_t * scale
    bq_s = (bq * scale).reshape(C, 1)
    wkv = jnp.concatenate([wk_t, wv_t], axis=1) * ln_g.reshape(C, 1)
    bkv = (jnp.concatenate([bk, bv])
           + jnp.dot(ln_b, jnp.concatenate([wk_t, wv_t], axis=1),
                     precision=jax.lax.Precision.HIGHEST)).reshape(1, 2 * C)

    bb = 4
    body = functools.partial(_fused_kernel, Hs=Hs, sr=sr, C=C, N=N,
                             tq=tq, eps=1e-5, bb=bb)

    return pl.pallas_call(
        body,
        out_shape=jax.ShapeDtypeStruct((B, N, C), x.dtype),
        grid=(B // bb,),
        in_specs=[
            pl.BlockSpec((bb, N, C), lambda b: (b, 0, 0)),
            pl.BlockSpec((bb, Hs, sr, Ws, sr * C), lambda b: (b, 0, 0, 0, 0)),
            pl.BlockSpec((sr * sr * C, C), lambda b: (0, 0)),
            pl.BlockSpec((1, C), lambda b: (0, 0)),
            pl.BlockSpec((C, C), lambda b: (0, 0)),
            pl.BlockSpec((C, 1), lambda b: (0, 0)),
            pl.BlockSpec((C, 2 * C), lambda b: (0, 0)),
            pl.BlockSpec((1, 2 * C), lambda b: (0, 0)),
            pl.BlockSpec((C, C), lambda b: (0, 0)),
            pl.BlockSpec((1, C), lambda b: (0, 0)),
        ],
        out_specs=pl.BlockSpec((bb, N, C), lambda b: (b, 0, 0)),
        compiler_params=pltpu.CompilerParams(
            dimension_semantics=("parallel",),
            vmem_limit_bytes=64 * 1024 * 1024,
        ),
    )(x, xp, wsr_r, bsr.reshape(1, C), wq_s, bq_s, wkv, bkv, wp_t,
      bp.reshape(1, C))
